# baseline (speedup 1.0000x reference)
import functools

import jax
import jax.numpy as jnp
import numpy as np
from jax import lax
from jax.experimental import pallas as pl
from jax.experimental.pallas import tpu as pltpu
from jax.experimental.pallas import tpu_sc as plsc

B = 4096
L = 50
VOCAB = 100000
D = 128

NC = 2
NS = 16
NW = NC * NS
NQ = 2
QSETS = B // NQ
BPW = QSETS // NW


SLAB = 128
NA = QSETS // SLAB
NB = NW // NA
LQ = -(-L // NB)
assert LQ % 2 == 1


def _sc_gather_body(qoff, table_hbm, xt_hbm, e_hbm,
                    slab_v, rows_a, rows_b, sga, sgb, swa, swb):
    wid = lax.axis_index("s") * NC + lax.axis_index("c")
    aw = lax.rem(wid, NA)
    bw = wid // NA
    setbase = aw * SLAB
    lbase = bw * LQ
    lim = jnp.minimum(lbase + LQ, L)
    pltpu.sync_copy(xt_hbm.at[:, pl.ds(qoff + setbase, SLAB)], slab_v)

    def fire_g(l, buf, sem):
        @pl.when(l < lim)
        def _():
            pltpu.async_copy(table_hbm.at[slab_v.at[l]], buf, sem)

    def wait_g(l, buf, sem):
        @pl.when(l < lim)
        def _():
            pltpu.make_async_copy(table_hbm.at[slab_v.at[l]], buf, sem).wait()

    def fire_w(l, buf, sem):
        @pl.when(l < lim)
        def _():
            pltpu.async_copy(buf, e_hbm.at[l, pl.ds(setbase, SLAB)], sem)

    def wait_w(l, buf, sem):
        @pl.when(l < lim)
        def _():
            pltpu.make_async_copy(buf,
                                  e_hbm.at[l, pl.ds(setbase, SLAB)],
                                  sem).wait()

    fire_g(lbase, rows_a, sga)
    fire_g(lbase + 1, rows_b, sgb)

    def pair(i, carry):
        la = lbase + 2 * i
        lb = la + 1
        wait_g(la, rows_a, sga)
        fire_w(la, rows_a, swa)
        wait_g(lb, rows_b, sgb)
        fire_w(lb, rows_b, swb)
        wait_w(la, rows_a, swa)
        fire_g(la + 2, rows_a, sga)
        wait_w(lb, rows_b, swb)
        fire_g(lb + 2, rows_b, sgb)
        return carry

    lax.fori_loop(0, LQ // 2, pair, 0)
    lt = lbase + LQ - 1
    wait_g(lt, rows_a, sga)
    fire_w(lt, rows_a, swa)
    wait_w(lt, rows_a, swa)


def _sc_gather(emb_table, xt, q):
    return pl.kernel(
        functools.partial(_sc_gather_body, q * QSETS),
        out_type=jax.ShapeDtypeStruct((L, QSETS, D), jnp.float32),
        mesh=plsc.VectorSubcoreMesh(core_axis_name="c", subcore_axis_name="s"),
        scratch_types=[
            pltpu.VMEM((L, SLAB), jnp.int32),
            pltpu.VMEM((SLAB, D), jnp.float32),
            pltpu.VMEM((SLAB, D), jnp.float32),
            pltpu.SemaphoreType.DMA,
            pltpu.SemaphoreType.DMA,
            pltpu.SemaphoreType.DMA,
            pltpu.SemaphoreType.DMA,
        ],
    )(emb_table, xt)


BT = 64
QBLOCKS = QSETS // BT


def _tc_compute(e_ref, w1_ref, w2_ref, o_ref):
    cdims = (((1,), (1,)), ((), ()))
    e3 = e_ref[...]
    m = jnp.max(e3, axis=0)
    e = e3.reshape(L * BT, D)
    m2 = lax.dot_general(m, w2_ref[...], cdims,
                         preferred_element_type=jnp.float32)
    e1 = lax.dot_general(e, w1_ref[...], cdims,
                         preferred_element_type=jnp.float32)
    c = e1.reshape(L, BT, D) + m2[None]
    o_ref[...] = 0.5 * jnp.tanh(0.5 * c) + 0.5


def _tc_body_first(e_ref, w1_ref, w2_ref, o_ref):
    _tc_compute(e_ref, w1_ref, w2_ref, o_ref)


def _tc_body_next(e_ref, w1_ref, w2_ref, o_prev_ref, o_ref):
    del o_prev_ref
    _tc_compute(e_ref, w1_ref, w2_ref, o_ref)


def _tc_stripe(e_q, W1, W2, q, o_prev):
    out_spec = pl.BlockSpec((L, BT, D), lambda i, q=q: (0, q * QBLOCKS + i, 0))
    in_specs = [
        pl.BlockSpec((L, BT, D), lambda i: (0, i, 0)),
        pl.BlockSpec((D, D), lambda i: (0, 0)),
        pl.BlockSpec((D, D), lambda i: (0, 0)),
    ]
    if o_prev is None:
        return pl.pallas_call(
            _tc_body_first,
            grid=(QBLOCKS,),
            in_specs=in_specs,
            out_specs=out_spec,
            out_shape=jax.ShapeDtypeStruct((L, B, D), jnp.float32),
        )(e_q, W1, W2)
    return pl.pallas_call(
        _tc_body_next,
        grid=(QBLOCKS,),
        in_specs=in_specs + [pl.BlockSpec(memory_space=pl.ANY)],
        out_specs=out_spec,
        out_shape=jax.ShapeDtypeStruct((L, B, D), jnp.float32),
        input_output_aliases={3: 0},
    )(e_q, W1, W2, o_prev)


def kernel(x, emb_table, W1, W2):
    xt = jnp.transpose(x.astype(jnp.int32), (1, 0))
    es = [_sc_gather(emb_table, xt, q) for q in range(NQ)]
    out = None
    for q in range(NQ):
        out = _tc_stripe(es[q], W1, W2, q, out)
    return jnp.transpose(out, (1, 0, 2))


# speedup vs baseline: 1.0135x; 1.0135x over previous
import functools

import jax
import jax.numpy as jnp
import numpy as np
from jax import lax
from jax.experimental import pallas as pl
from jax.experimental.pallas import tpu as pltpu
from jax.experimental.pallas import tpu_sc as plsc

B = 4096
L = 50
VOCAB = 100000
D = 128

NC = 2
NS = 16
NW = NC * NS
NQ = 4
QSETS = B // NQ
BPW = QSETS // NW


SLAB = 128
NA = QSETS // SLAB
NB = NW // NA
LQ = -(-L // NB)
assert LQ % 2 == 1


def _sc_gather_body(qoff, table_hbm, xt_hbm, e_hbm,
                    slab_v, rows_a, rows_b, sga, sgb, swa, swb):
    wid = lax.axis_index("s") * NC + lax.axis_index("c")
    aw = lax.rem(wid, NA)
    bw = wid // NA
    setbase = aw * SLAB
    lbase = bw * LQ
    lim = jnp.minimum(lbase + LQ, L)
    pltpu.sync_copy(xt_hbm.at[:, pl.ds(qoff + setbase, SLAB)], slab_v)

    def fire_g(l, buf, sem):
        @pl.when(l < lim)
        def _():
            pltpu.async_copy(table_hbm.at[slab_v.at[l]], buf, sem)

    def wait_g(l, buf, sem):
        @pl.when(l < lim)
        def _():
            pltpu.make_async_copy(table_hbm.at[slab_v.at[l]], buf, sem).wait()

    def fire_w(l, buf, sem):
        @pl.when(l < lim)
        def _():
            pltpu.async_copy(buf, e_hbm.at[l, pl.ds(setbase, SLAB)], sem)

    def wait_w(l, buf, sem):
        @pl.when(l < lim)
        def _():
            pltpu.make_async_copy(buf,
                                  e_hbm.at[l, pl.ds(setbase, SLAB)],
                                  sem).wait()

    fire_g(lbase, rows_a, sga)
    fire_g(lbase + 1, rows_b, sgb)

    def pair(i, carry):
        la = lbase + 2 * i
        lb = la + 1
        wait_g(la, rows_a, sga)
        fire_w(la, rows_a, swa)
        wait_g(lb, rows_b, sgb)
        fire_w(lb, rows_b, swb)
        wait_w(la, rows_a, swa)
        fire_g(la + 2, rows_a, sga)
        wait_w(lb, rows_b, swb)
        fire_g(lb + 2, rows_b, sgb)
        return carry

    lax.fori_loop(0, LQ // 2, pair, 0)
    lt = lbase + LQ - 1
    wait_g(lt, rows_a, sga)
    fire_w(lt, rows_a, swa)
    wait_w(lt, rows_a, swa)


def _sc_gather(emb_table, xt, q):
    return pl.kernel(
        functools.partial(_sc_gather_body, q * QSETS),
        out_type=jax.ShapeDtypeStruct((L, QSETS, D), jnp.float32),
        mesh=plsc.VectorSubcoreMesh(core_axis_name="c", subcore_axis_name="s"),
        scratch_types=[
            pltpu.VMEM((L, SLAB), jnp.int32),
            pltpu.VMEM((SLAB, D), jnp.float32),
            pltpu.VMEM((SLAB, D), jnp.float32),
            pltpu.SemaphoreType.DMA,
            pltpu.SemaphoreType.DMA,
            pltpu.SemaphoreType.DMA,
            pltpu.SemaphoreType.DMA,
        ],
    )(emb_table, xt)


BT = 128
QBLOCKS = QSETS // BT


def _tc_compute(e_ref, w1_ref, w2_ref, o_ref):
    cdims = (((1,), (1,)), ((), ()))
    e3 = e_ref[...]
    m = jnp.max(e3, axis=0)
    e = e3.reshape(L * BT, D)
    m2 = lax.dot_general(m, w2_ref[...], cdims,
                         preferred_element_type=jnp.float32)
    e1 = lax.dot_general(e, w1_ref[...], cdims,
                         preferred_element_type=jnp.float32)
    c = e1.reshape(L, BT, D) + m2[None]
    o_ref[...] = 0.5 * jnp.tanh(0.5 * c) + 0.5


def _tc_body_first(e_ref, w1_ref, w2_ref, o_ref):
    _tc_compute(e_ref, w1_ref, w2_ref, o_ref)


def _tc_body_next(e_ref, w1_ref, w2_ref, o_prev_ref, o_ref):
    del o_prev_ref
    _tc_compute(e_ref, w1_ref, w2_ref, o_ref)


def _tc_stripe(e_q, W1, W2, q, o_prev):
    out_spec = pl.BlockSpec((L, BT, D), lambda i, q=q: (0, q * QBLOCKS + i, 0))
    in_specs = [
        pl.BlockSpec((L, BT, D), lambda i: (0, i, 0)),
        pl.BlockSpec((D, D), lambda i: (0, 0)),
        pl.BlockSpec((D, D), lambda i: (0, 0)),
    ]
    if o_prev is None:
        return pl.pallas_call(
            _tc_body_first,
            grid=(QBLOCKS,),
            in_specs=in_specs,
            out_specs=out_spec,
            out_shape=jax.ShapeDtypeStruct((L, B, D), jnp.float32),
        )(e_q, W1, W2)
    return pl.pallas_call(
        _tc_body_next,
        grid=(QBLOCKS,),
        in_specs=in_specs + [pl.BlockSpec(memory_space=pl.ANY)],
        out_specs=out_spec,
        out_shape=jax.ShapeDtypeStruct((L, B, D), jnp.float32),
        input_output_aliases={3: 0},
    )(e_q, W1, W2, o_prev)


def kernel(x, emb_table, W1, W2):
    xt = jnp.transpose(x.astype(jnp.int32), (1, 0))
    es = [_sc_gather(emb_table, xt, q) for q in range(NQ)]
    out = None
    for q in range(NQ):
        out = _tc_stripe(es[q], W1, W2, q, out)
    return jnp.transpose(out, (1, 0, 2))
